# Initial kernel scaffold; baseline (speedup 1.0000x reference)
import jax
import jax.numpy as jnp
from jax.experimental import pallas as pl


def kernel(x, adj, W_0_0, b_0_0, W_0_1, b_0_1, W_1_0, b_1_0, W_1_1, b_1_1, W_out, b_out):
    raise NotImplementedError("write your pallas kernel here")


# speedup vs baseline: 3.0796x; 3.0796x over previous
import jax
import jax.numpy as jnp
from jax.experimental import pallas as pl
from jax.experimental.pallas import tpu as pltpu

_THRESH = 0.01
_S = 1024
_D = 256
_M = 2


def _gconv_body(x_ref, adj_ref, w00_ref, w01_ref, w10_ref, w11_ref,
                b0_ref, b1_ref, wo_ref, bo_ref, out_ref, ah_ref):
    b = pl.program_id(0)

    @pl.when(b == 0)
    def _build_ah():
        rows = jax.lax.broadcasted_iota(jnp.int32, (_S, _S), 0)
        cols = jax.lax.broadcasted_iota(jnp.int32, (_S, _S), 1)
        eye = jnp.where(rows == cols, jnp.float32(1.0), jnp.float32(0.0))
        for m in range(_M):
            a = adj_ref[m]
            sym = jnp.where(rows >= cols, a, a.T)
            sa = jnp.abs(sym)
            c = jnp.where(sa > _THRESH, sa, jnp.float32(0.0))
            deg = jnp.sum(c, axis=1) + 1.0
            dinv = jnp.where(deg > 0.0, jax.lax.rsqrt(deg), jnp.float32(0.0))
            ah_ref[m] = dinv[:, None] * (c + eye) * dinv[None, :]

    xb = x_ref[0]
    ws0 = w00_ref[:] + w01_ref[:]
    ws1 = w10_ref[:] + w11_ref[:]
    h0 = jnp.dot(xb, ws0, preferred_element_type=jnp.float32)
    h1 = jnp.dot(xb, ws1, preferred_element_type=jnp.float32)
    y0 = jnp.dot(ah_ref[0], h0, preferred_element_type=jnp.float32) + b0_ref[0][None, :]
    y1 = jnp.dot(ah_ref[1], h1, preferred_element_type=jnp.float32) + b1_ref[0][None, :]
    y0 = jnp.maximum(y0, 0.0)
    y1 = jnp.maximum(y1, 0.0)
    o = jnp.dot(y0, wo_ref[0:_D], preferred_element_type=jnp.float32)
    o += jnp.dot(y1, wo_ref[_D:2 * _D], preferred_element_type=jnp.float32)
    o += bo_ref[0][None, :]
    out_ref[0] = jnp.maximum(o, 0.0)


def kernel(x, adj, W_0_0, b_0_0, W_0_1, b_0_1, W_1_0, b_1_0, W_1_1, b_1_1,
           W_out, b_out):
    B = x.shape[0]
    b0 = (b_0_0 + b_0_1).reshape(1, _D)
    b1 = (b_1_0 + b_1_1).reshape(1, _D)
    bo = b_out.reshape(1, _D)
    const = lambda *_: (0,) * 3
    const2 = lambda *_: (0, 0)
    return pl.pallas_call(
        _gconv_body,
        grid=(B,),
        in_specs=[
            pl.BlockSpec((1, _S, _D), lambda b: (b, 0, 0)),
            pl.BlockSpec((_M, _S, _S), const),
            pl.BlockSpec((_D, _D), const2),
            pl.BlockSpec((_D, _D), const2),
            pl.BlockSpec((_D, _D), const2),
            pl.BlockSpec((_D, _D), const2),
            pl.BlockSpec((1, _D), const2),
            pl.BlockSpec((1, _D), const2),
            pl.BlockSpec((_M * _D, _D), const2),
            pl.BlockSpec((1, _D), const2),
        ],
        out_specs=pl.BlockSpec((1, _S, _D), lambda b: (b, 0, 0)),
        out_shape=jax.ShapeDtypeStruct((B, _S, _D), jnp.float32),
        scratch_shapes=[pltpu.VMEM((_M, _S, _S), jnp.float32)],
    )(x, adj, W_0_0, W_0_1, W_1_0, W_1_1, b0, b1, W_out, bo)
